# Initial kernel scaffold; baseline (speedup 1.0000x reference)
import jax
import jax.numpy as jnp
from jax.experimental import pallas as pl


def kernel(coords, smpl_V, smpl_F, can_V):
    raise NotImplementedError("write your pallas kernel here")


# speedup vs baseline: 4.4525x; 4.4525x over previous
import functools

import jax
import jax.numpy as jnp
from jax import lax
from jax.experimental import pallas as pl
from jax.experimental.pallas import tpu as pltpu
from jax.experimental.pallas import tpu_sc as plsc

B = 2
S = 4096
V = 6890
F = 13776
FP = 13824
NT = FP // 128
PT = 8
NW = 32
FCH = FP // NW
BS = B * S
PCH = BS // NW
IMAX = 2147483647
BIG = 3e38
PADD = 1e30


def _sc_mesh():
    return plsc.VectorSubcoreMesh(core_axis_name="c", subcore_axis_name="s")


def _wid():
    return lax.axis_index("s") * 2 + lax.axis_index("c")


def _tri_gather_body(c0_h, c1_h, c2_h, svx_h, svy_h, svz_h, *refs):
    outs = refs[:B * 9]
    c0v, c1v, c2v, svx, svy, svz = refs[B * 9:B * 9 + 6]
    fb = refs[B * 9 + 6:]
    w = _wid()
    base = w * FCH
    pltpu.sync_copy(c0_h.at[pl.ds(base, FCH)], c0v)
    pltpu.sync_copy(c1_h.at[pl.ds(base, FCH)], c1v)
    pltpu.sync_copy(c2_h.at[pl.ds(base, FCH)], c2v)
    pltpu.sync_copy(svx_h, svx)
    pltpu.sync_copy(svy_h, svy)
    pltpu.sync_copy(svz_h, svz)
    for g in range(FCH // 16):
        sl = pl.ds(g * 16, 16)
        i0 = c0v[sl]
        i1 = c1v[sl]
        i2 = c2v[sl]
        for b in range(B):
            a0 = i0 + b * V
            a1 = i1 + b * V
            a2 = i2 + b * V
            fb[b * 9 + 0][sl] = plsc.load_gather(svx, [a0])
            fb[b * 9 + 1][sl] = plsc.load_gather(svy, [a0])
            fb[b * 9 + 2][sl] = plsc.load_gather(svz, [a0])
            fb[b * 9 + 3][sl] = plsc.load_gather(svx, [a1])
            fb[b * 9 + 4][sl] = plsc.load_gather(svy, [a1])
            fb[b * 9 + 5][sl] = plsc.load_gather(svz, [a1])
            fb[b * 9 + 6][sl] = plsc.load_gather(svx, [a2])
            fb[b * 9 + 7][sl] = plsc.load_gather(svy, [a2])
            fb[b * 9 + 8][sl] = plsc.load_gather(svz, [a2])
    for b in range(B):
        for r in range(9):
            pltpu.sync_copy(fb[b * 9 + r], outs[b * 9 + r].at[pl.ds(base, FCH)])


@functools.cache
def _tri_gather_kernel():
    return pl.kernel(
        _tri_gather_body,
        mesh=_sc_mesh(),
        compiler_params=pltpu.CompilerParams(needs_layout_passes=False),
        out_type=[jax.ShapeDtypeStruct((FP,), jnp.float32)] * (B * 9),
        scratch_types=[
            pltpu.VMEM((FCH,), jnp.int32),
            pltpu.VMEM((FCH,), jnp.int32),
            pltpu.VMEM((FCH,), jnp.int32),
            pltpu.VMEM((B * V,), jnp.float32),
            pltpu.VMEM((B * V,), jnp.float32),
            pltpu.VMEM((B * V,), jnp.float32),
        ] + [pltpu.VMEM((FCH,), jnp.float32)] * (B * 9),
    )


def _tri_gather(*args):
    return _tri_gather_kernel()(*args)


def _sweep_body(cref, tref, oref):
    c8 = cref[0, 0]
    px = c8[:, 0:1]
    py = c8[:, 1:2]
    pz = c8[:, 2:3]
    lane = lax.broadcasted_iota(jnp.int32, (PT, 128), 1)
    zero = jnp.zeros((PT, 128), jnp.float32)
    tiny = jnp.float32(1e-12)
    one = jnp.float32(1.0)

    def body(t, carry):
        min1, fid1, min2, fid2 = carry
        tb = tref[0, t]
        ax = tb[0:1]; ay = tb[1:2]; az = tb[2:3]
        bx = tb[3:4]; by = tb[4:5]; bz = tb[5:6]
        cx = tb[6:7]; cy = tb[7:8]; cz = tb[8:9]
        abx = bx - ax; aby = by - ay; abz = bz - az
        acx = cx - ax; acy = cy - ay; acz = cz - az
        bb = abx * abx + aby * aby + abz * abz
        cc = acx * acx + acy * acy + acz * acz
        bcd = abx * acx + aby * acy + abz * acz
        nn = bb * cc - bcd * bcd
        ibb = jnp.where(bb > tiny, one / bb, one)
        icc = jnp.where(cc > tiny, one / cc, one)
        bcbc = bb - 2.0 * bcd + cc
        ibcbc = jnp.where(bcbc > tiny, one / bcbc, one)
        inn = jnp.where(nn > tiny, one / nn, one)

        apx = px - ax
        apy = py - ay
        apz = pz - az
        s1 = abx * apx + aby * apy + abz * apz
        s2 = acx * apx + acy * apy + acz * apz
        s3 = s1 - bb
        s4 = s2 - bcd
        s5 = s1 - bcd
        s6 = s2 - cc
        vc = bb * s2 - bcd * s1
        vb = cc * s1 - bcd * s2
        va = nn - vb - vc
        v = vb * inn
        w = vc * inn
        h1 = s4 - s3
        h2 = s5 - s6
        t6 = h1 * ibcbc
        c6 = (va <= 0.0) & (h1 >= 0.0) & (h2 >= 0.0)
        v = jnp.where(c6, one - t6, v); w = jnp.where(c6, t6, w)
        t5 = s2 * icc
        c5 = (vb <= 0.0) & (s2 >= 0.0) & (s6 <= 0.0)
        v = jnp.where(c5, zero, v); w = jnp.where(c5, t5, w)
        t4 = s1 * ibb
        c4 = (vc <= 0.0) & (s1 >= 0.0) & (s3 <= 0.0)
        v = jnp.where(c4, t4, v); w = jnp.where(c4, zero, w)
        c3 = (s6 >= 0.0) & (s5 <= s6)
        v = jnp.where(c3, zero, v); w = jnp.where(c3, one + zero, w)
        c2 = (s3 >= 0.0) & (s4 <= s3)
        v = jnp.where(c2, one + zero, v); w = jnp.where(c2, zero, w)
        c1 = (s1 <= 0.0) & (s2 <= 0.0)
        v = jnp.where(c1, zero, v); w = jnp.where(c1, zero, w)
        dvx = apx - v * abx - w * acx
        dvy = apy - v * aby - w * acy
        dvz = apz - v * abz - w * acz
        dd = dvx * dvx + dvy * dvy + dvz * dvz
        fvec = lane + t * 128
        dd = jnp.where(fvec < F, dd, PADD)
        lt1 = dd < min1
        lt2 = (dd < min2) & (~lt1)
        min2 = jnp.where(lt1, min1, jnp.where(lt2, dd, min2))
        fid2 = jnp.where(lt1, fid1, jnp.where(lt2, fvec, fid2))
        min1 = jnp.where(lt1, dd, min1)
        fid1 = jnp.where(lt1, fvec, fid1)
        return min1, fid1, min2, fid2

    init = (jnp.full((PT, 128), BIG), jnp.zeros((PT, 128), jnp.int32),
            jnp.full((PT, 128), BIG), jnp.zeros((PT, 128), jnp.int32))
    min1, fid1, min2, fid2 = lax.fori_loop(0, NT, body, init)
    m1 = jnp.min(min1, axis=1, keepdims=True)
    sel1 = min1 == m1
    f1 = jnp.min(jnp.where(sel1, fid1, IMAX), axis=1, keepdims=True)
    on1 = fid1 == f1
    cand2 = jnp.where(on1, min2, min1)
    cf2 = jnp.where(on1, fid2, fid1)
    m2 = jnp.min(cand2, axis=1, keepdims=True)
    f2 = jnp.min(jnp.where(cand2 == m2, cf2, IMAX), axis=1, keepdims=True)
    oref[0, 0, :, 0:1] = f1
    oref[0, 0, :, 1:2] = f2


def _top2_faces(coords4, t9):
    return pl.pallas_call(
        _sweep_body,
        grid=(B, S // PT),
        in_specs=[
            pl.BlockSpec((1, 1, PT, 3), lambda b, s: (b, s, 0, 0)),
            pl.BlockSpec((1, NT, 9, 128), lambda b, s: (b, 0, 0, 0)),
        ],
        out_specs=pl.BlockSpec((1, 1, PT, 2), lambda b, s: (b, s, 0, 0)),
        out_shape=jax.ShapeDtypeStruct((B, S // PT, PT, 2), jnp.int32),
    )(coords4, t9)


def _cand_gather_body(f1_h, f2_h, c0_h, c1_h, c2_h, svx_h, svy_h, svz_h,
                      cvx_h, cvy_h, cvz_h, *refs):
    outs = refs[:36]
    (f1v, f2v, c0v, c1v, c2v, svx, svy, svz, cvx, cvy, cvz) = refs[36:47]
    fb = refs[47:]
    w = _wid()
    base = w * PCH
    boff = (w // 16) * V
    pltpu.sync_copy(f1_h.at[pl.ds(base, PCH)], f1v)
    pltpu.sync_copy(f2_h.at[pl.ds(base, PCH)], f2v)
    pltpu.sync_copy(c0_h, c0v)
    pltpu.sync_copy(c1_h, c1v)
    pltpu.sync_copy(c2_h, c2v)
    pltpu.sync_copy(svx_h, svx)
    pltpu.sync_copy(svy_h, svy)
    pltpu.sync_copy(svz_h, svz)
    pltpu.sync_copy(cvx_h, cvx)
    pltpu.sync_copy(cvy_h, cvy)
    pltpu.sync_copy(cvz_h, cvz)
    for g in range(PCH // 16):
        sl = pl.ds(g * 16, 16)
        for k, fv in ((0, f1v), (1, f2v)):
            fk = fv[sl]
            i0 = plsc.load_gather(c0v, [fk])
            i1 = plsc.load_gather(c1v, [fk])
            i2 = plsc.load_gather(c2v, [fk])
            a0 = i0 + boff
            a1 = i1 + boff
            a2 = i2 + boff
            r = k * 18
            fb[r + 0][sl] = plsc.load_gather(svx, [a0])
            fb[r + 1][sl] = plsc.load_gather(svy, [a0])
            fb[r + 2][sl] = plsc.load_gather(svz, [a0])
            fb[r + 3][sl] = plsc.load_gather(svx, [a1])
            fb[r + 4][sl] = plsc.load_gather(svy, [a1])
            fb[r + 5][sl] = plsc.load_gather(svz, [a1])
            fb[r + 6][sl] = plsc.load_gather(svx, [a2])
            fb[r + 7][sl] = plsc.load_gather(svy, [a2])
            fb[r + 8][sl] = plsc.load_gather(svz, [a2])
            fb[r + 9][sl] = plsc.load_gather(cvx, [i0])
            fb[r + 10][sl] = plsc.load_gather(cvy, [i0])
            fb[r + 11][sl] = plsc.load_gather(cvz, [i0])
            fb[r + 12][sl] = plsc.load_gather(cvx, [i1])
            fb[r + 13][sl] = plsc.load_gather(cvy, [i1])
            fb[r + 14][sl] = plsc.load_gather(cvz, [i1])
            fb[r + 15][sl] = plsc.load_gather(cvx, [i2])
            fb[r + 16][sl] = plsc.load_gather(cvy, [i2])
            fb[r + 17][sl] = plsc.load_gather(cvz, [i2])
    for r in range(36):
        pltpu.sync_copy(fb[r], outs[r].at[pl.ds(base, PCH)])


@functools.cache
def _cand_gather_kernel():
    return pl.kernel(
        _cand_gather_body,
        mesh=_sc_mesh(),
        compiler_params=pltpu.CompilerParams(needs_layout_passes=False),
        out_type=[jax.ShapeDtypeStruct((BS,), jnp.float32)] * 36,
        scratch_types=[
            pltpu.VMEM((PCH,), jnp.int32),
            pltpu.VMEM((PCH,), jnp.int32),
            pltpu.VMEM((FP,), jnp.int32),
            pltpu.VMEM((FP,), jnp.int32),
            pltpu.VMEM((FP,), jnp.int32),
            pltpu.VMEM((B * V,), jnp.float32),
            pltpu.VMEM((B * V,), jnp.float32),
            pltpu.VMEM((B * V,), jnp.float32),
            pltpu.VMEM((V,), jnp.float32),
            pltpu.VMEM((V,), jnp.float32),
            pltpu.VMEM((V,), jnp.float32),
        ] + [pltpu.VMEM((PCH,), jnp.float32)] * 36,
    )


def _cand_gather(*args):
    return _cand_gather_kernel()(*args)


def _sdiv(x, y):
    return x / jnp.where(jnp.abs(y) > 1e-12, y, 1.0)


def _eric_verbatim(px, py, pz, tb, r0):
    ax = tb[r0 + 0:r0 + 1]; ay = tb[r0 + 1:r0 + 2]; az = tb[r0 + 2:r0 + 3]
    bx = tb[r0 + 3:r0 + 4]; by = tb[r0 + 4:r0 + 5]; bz = tb[r0 + 5:r0 + 6]
    cx = tb[r0 + 6:r0 + 7]; cy = tb[r0 + 7:r0 + 8]; cz = tb[r0 + 8:r0 + 9]
    abx = bx - ax; aby = by - ay; abz = bz - az
    acx = cx - ax; acy = cy - ay; acz = cz - az
    apx = px - ax; apy = py - ay; apz = pz - az
    d1 = abx * apx + aby * apy + abz * apz
    d2 = acx * apx + acy * apy + acz * apz
    bpx = px - bx; bpy = py - by; bpz = pz - bz
    d3 = abx * bpx + aby * bpy + abz * bpz
    d4 = acx * bpx + acy * bpy + acz * bpz
    cpx = px - cx; cpy = py - cy; cpz = pz - cz
    d5 = abx * cpx + aby * cpy + abz * cpz
    d6 = acx * cpx + acy * cpy + acz * cpz
    vc = d1 * d4 - d3 * d2
    vb = d5 * d2 - d1 * d6
    va = d3 * d6 - d5 * d4
    denom = va + vb + vc
    v_i = _sdiv(vb, denom)
    w_i = _sdiv(vc, denom)
    u = 1.0 - v_i - w_i
    v = v_i
    w = w_i
    t6 = _sdiv(d4 - d3, (d4 - d3) + (d5 - d6))
    c6 = (va <= 0.0) & ((d4 - d3) >= 0.0) & ((d5 - d6) >= 0.0)
    u = jnp.where(c6, 0.0, u); v = jnp.where(c6, 1.0 - t6, v)
    w = jnp.where(c6, t6, w)
    t5 = _sdiv(d2, d2 - d6)
    c5 = (vb <= 0.0) & (d2 >= 0.0) & (d6 <= 0.0)
    u = jnp.where(c5, 1.0 - t5, u); v = jnp.where(c5, 0.0, v)
    w = jnp.where(c5, t5, w)
    t4 = _sdiv(d1, d1 - d3)
    c4 = (vc <= 0.0) & (d1 >= 0.0) & (d3 <= 0.0)
    u = jnp.where(c4, 1.0 - t4, u); v = jnp.where(c4, t4, v)
    w = jnp.where(c4, 0.0, w)
    c3 = (d6 >= 0.0) & (d5 <= d6)
    u = jnp.where(c3, 0.0, u); v = jnp.where(c3, 0.0, v)
    w = jnp.where(c3, 1.0, w)
    c2 = (d3 >= 0.0) & (d4 <= d3)
    u = jnp.where(c2, 0.0, u); v = jnp.where(c2, 1.0, v)
    w = jnp.where(c2, 0.0, w)
    c1 = (d1 <= 0.0) & (d2 <= 0.0)
    u = jnp.where(c1, 1.0, u); v = jnp.where(c1, 0.0, v)
    w = jnp.where(c1, 0.0, w)
    cptx = ax * u + bx * v + cx * w
    cpty = ay * u + by * v + cy * w
    cptz = az * u + bz * v + cz * w
    dx = px - cptx; dy = py - cpty; dz = pz - cptz
    dd = dx * dx + dy * dy + dz * dz
    return u, v, w, dd, cptx, cpty, cptz


def _final_body(gref, fref, cref, oref):
    tb = gref[...]
    px = cref[0:1]; py = cref[1:2]; pz = cref[2:3]
    f1 = fref[0:1]; f2 = fref[1:2]
    u1, v1, w1, dd1, x1, y1, z1 = _eric_verbatim(px, py, pz, tb, 0)
    u2, v2, w2, dd2, x2, y2, z2 = _eric_verbatim(px, py, pz, tb, 18)
    awin = dd1 <= dd2
    u = jnp.where(awin, u1, u2); v = jnp.where(awin, v1, v2)
    w = jnp.where(awin, w1, w2)
    dd = jnp.where(awin, dd1, dd2)
    hx = jnp.where(awin, x1, x2); hy = jnp.where(awin, y1, y2)
    hz = jnp.where(awin, z1, z2)
    cax = jnp.where(awin, tb[9:10], tb[27:28])
    cay = jnp.where(awin, tb[10:11], tb[28:29])
    caz = jnp.where(awin, tb[11:12], tb[29:30])
    cbx = jnp.where(awin, tb[12:13], tb[30:31])
    cby = jnp.where(awin, tb[13:14], tb[31:32])
    cbz = jnp.where(awin, tb[14:15], tb[32:33])
    ccx = jnp.where(awin, tb[15:16], tb[33:34])
    ccy = jnp.where(awin, tb[16:17], tb[34:35])
    ccz = jnp.where(awin, tb[17:18], tb[35:36])
    ox = cax * u + cbx * v + ccx * w
    oy = cay * u + cby * v + ccy * w
    oz = caz * u + cbz * v + ccz * w
    sdf = jnp.sqrt(jnp.maximum(dd, 1e-12))
    dx = hx - px; dy = hy - py; dz = hz - pz
    nrm = jnp.sqrt(dx * dx + dy * dy + dz * dz)
    nd = jnp.maximum(nrm, 1e-6)
    oref[0:1] = ox
    oref[1:2] = oy
    oref[2:3] = oz
    oref[3:4] = sdf
    oref[4:5] = dx / nd
    oref[5:6] = dy / nd
    oref[6:7] = dz / nd
    oref[7:8] = nrm


def _finalize(g40, fids, ct):
    tl = 1024
    return pl.pallas_call(
        _final_body,
        grid=(BS // tl,),
        in_specs=[
            pl.BlockSpec((40, tl), lambda i: (0, i)),
            pl.BlockSpec((8, tl), lambda i: (0, i)),
            pl.BlockSpec((8, tl), lambda i: (0, i)),
        ],
        out_specs=pl.BlockSpec((8, tl), lambda i: (0, i)),
        out_shape=jax.ShapeDtypeStruct((8, BS), jnp.float32),
    )(g40, fids, ct)


def kernel(coords, smpl_V, smpl_F, can_V):
    coords = coords.astype(jnp.float32)
    smpl_V = smpl_V.astype(jnp.float32)
    can_V = can_V.astype(jnp.float32)
    sfp = jnp.pad(smpl_F, ((0, FP - F), (0, 0)))
    c0 = sfp[:, 0].astype(jnp.int32)
    c1 = sfp[:, 1].astype(jnp.int32)
    c2 = sfp[:, 2].astype(jnp.int32)
    svx = smpl_V[:, :, 0].reshape(B * V)
    svy = smpl_V[:, :, 1].reshape(B * V)
    svz = smpl_V[:, :, 2].reshape(B * V)
    cvx = can_V[:, 0]
    cvy = can_V[:, 1]
    cvz = can_V[:, 2]

    tri_rows = _tri_gather(c0, c1, c2, svx, svy, svz)
    tri9 = jnp.stack(tri_rows).reshape(B, 9, FP)
    t9 = tri9.reshape(B, 9, NT, 128).transpose(0, 2, 1, 3)
    coords4 = coords.reshape(B, S // PT, PT, 3)
    fid2 = _top2_faces(coords4, t9)
    fflat = fid2.reshape(BS, 2)
    f1 = fflat[:, 0]
    f2 = fflat[:, 1]
    g_rows = _cand_gather(f1, f2, c0, c1, c2, svx, svy, svz, cvx, cvy, cvz)
    g40 = jnp.concatenate(
        [jnp.stack(g_rows), jnp.zeros((4, BS), jnp.float32)], axis=0)
    fids = jnp.zeros((8, BS), jnp.int32).at[0].set(f1).at[1].set(f2)
    ct = jnp.zeros((8, BS), jnp.float32)
    ct = ct.at[0].set(coords[..., 0].reshape(BS))
    ct = ct.at[1].set(coords[..., 1].reshape(BS))
    ct = ct.at[2].set(coords[..., 2].reshape(BS))
    out8 = _finalize(g40, fids, ct)
    out_coord = out8[0:3].T.reshape(B, S, 3)
    sdf = out8[3].reshape(B, S)
    normal = out8[4:7].T.reshape(B, S, 3)
    z = coords[..., 2:3]
    return (out_coord, sdf, normal, z)


# speedup vs baseline: 12.5987x; 2.8296x over previous
import functools

import jax
import jax.numpy as jnp
from jax import lax
from jax.experimental import pallas as pl
from jax.experimental.pallas import tpu as pltpu
from jax.experimental.pallas import tpu_sc as plsc

B = 2
S = 4096
V = 6890
F = 13776
FP = 13824
NT = FP // 128
PT = 32
NW = 32
FCH = FP // NW
BS = B * S
PCH = BS // NW
IMAX = 2147483647
BIG = 3e38
PADD = 1e30


def _sc_mesh():
    return plsc.VectorSubcoreMesh(core_axis_name="c", subcore_axis_name="s")


def _wid():
    return lax.axis_index("s") * 2 + lax.axis_index("c")


def _tri_gather_body(c0_h, c1_h, c2_h, svx_h, svy_h, svz_h, *refs):
    outs = refs[:B * 9]
    c0v, c1v, c2v, svx, svy, svz = refs[B * 9:B * 9 + 6]
    fb = refs[B * 9 + 6:]
    w = _wid()
    base = w * FCH
    pltpu.sync_copy(c0_h.at[pl.ds(base, FCH)], c0v)
    pltpu.sync_copy(c1_h.at[pl.ds(base, FCH)], c1v)
    pltpu.sync_copy(c2_h.at[pl.ds(base, FCH)], c2v)
    pltpu.sync_copy(svx_h, svx)
    pltpu.sync_copy(svy_h, svy)
    pltpu.sync_copy(svz_h, svz)
    for g in range(FCH // 16):
        sl = pl.ds(g * 16, 16)
        i0 = c0v[sl]
        i1 = c1v[sl]
        i2 = c2v[sl]
        for b in range(B):
            a0 = i0 + b * V
            a1 = i1 + b * V
            a2 = i2 + b * V
            fb[b * 9 + 0][sl] = plsc.load_gather(svx, [a0])
            fb[b * 9 + 1][sl] = plsc.load_gather(svy, [a0])
            fb[b * 9 + 2][sl] = plsc.load_gather(svz, [a0])
            fb[b * 9 + 3][sl] = plsc.load_gather(svx, [a1])
            fb[b * 9 + 4][sl] = plsc.load_gather(svy, [a1])
            fb[b * 9 + 5][sl] = plsc.load_gather(svz, [a1])
            fb[b * 9 + 6][sl] = plsc.load_gather(svx, [a2])
            fb[b * 9 + 7][sl] = plsc.load_gather(svy, [a2])
            fb[b * 9 + 8][sl] = plsc.load_gather(svz, [a2])
    for b in range(B):
        for r in range(9):
            pltpu.sync_copy(fb[b * 9 + r], outs[b * 9 + r].at[pl.ds(base, FCH)])


@functools.cache
def _tri_gather_kernel():
    return pl.kernel(
        _tri_gather_body,
        mesh=_sc_mesh(),
        compiler_params=pltpu.CompilerParams(needs_layout_passes=False),
        out_type=[jax.ShapeDtypeStruct((FP,), jnp.float32)] * (B * 9),
        scratch_types=[
            pltpu.VMEM((FCH,), jnp.int32),
            pltpu.VMEM((FCH,), jnp.int32),
            pltpu.VMEM((FCH,), jnp.int32),
            pltpu.VMEM((B * V,), jnp.float32),
            pltpu.VMEM((B * V,), jnp.float32),
            pltpu.VMEM((B * V,), jnp.float32),
        ] + [pltpu.VMEM((FCH,), jnp.float32)] * (B * 9),
    )


def _tri_gather(*args):
    return _tri_gather_kernel()(*args)


def _sweep_body(cref, tref, oref):
    c8 = cref[0, 0]
    px = c8[:, 0:1]
    py = c8[:, 1:2]
    pz = c8[:, 2:3]
    lane = lax.broadcasted_iota(jnp.int32, (PT, 128), 1)
    zero = jnp.zeros((PT, 128), jnp.float32)
    tiny = jnp.float32(1e-12)
    one = jnp.float32(1.0)

    def body(t, carry):
        min1, fid1, min2, fid2 = carry
        tb = tref[0, t]
        ax = tb[0:1]; ay = tb[1:2]; az = tb[2:3]
        bx = tb[3:4]; by = tb[4:5]; bz = tb[5:6]
        cx = tb[6:7]; cy = tb[7:8]; cz = tb[8:9]
        abx = bx - ax; aby = by - ay; abz = bz - az
        acx = cx - ax; acy = cy - ay; acz = cz - az
        bb = abx * abx + aby * aby + abz * abz
        cc = acx * acx + acy * acy + acz * acz
        bcd = abx * acx + aby * acy + abz * acz
        nn = bb * cc - bcd * bcd
        ibb = jnp.where(bb > tiny, one / bb, one)
        icc = jnp.where(cc > tiny, one / cc, one)
        bcbc = bb - 2.0 * bcd + cc
        ibcbc = jnp.where(bcbc > tiny, one / bcbc, one)
        inn = jnp.where(nn > tiny, one / nn, one)

        apx = px - ax
        apy = py - ay
        apz = pz - az
        s1 = abx * apx + aby * apy + abz * apz
        s2 = acx * apx + acy * apy + acz * apz
        s3 = s1 - bb
        s4 = s2 - bcd
        s5 = s1 - bcd
        s6 = s2 - cc
        vc = bb * s2 - bcd * s1
        vb = cc * s1 - bcd * s2
        va = nn - vb - vc
        v = vb * inn
        w = vc * inn
        h1 = s4 - s3
        h2 = s5 - s6
        t6 = h1 * ibcbc
        c6 = (va <= 0.0) & (h1 >= 0.0) & (h2 >= 0.0)
        v = jnp.where(c6, one - t6, v); w = jnp.where(c6, t6, w)
        t5 = s2 * icc
        c5 = (vb <= 0.0) & (s2 >= 0.0) & (s6 <= 0.0)
        v = jnp.where(c5, zero, v); w = jnp.where(c5, t5, w)
        t4 = s1 * ibb
        c4 = (vc <= 0.0) & (s1 >= 0.0) & (s3 <= 0.0)
        v = jnp.where(c4, t4, v); w = jnp.where(c4, zero, w)
        c3 = (s6 >= 0.0) & (s5 <= s6)
        v = jnp.where(c3, zero, v); w = jnp.where(c3, one + zero, w)
        c2 = (s3 >= 0.0) & (s4 <= s3)
        v = jnp.where(c2, one + zero, v); w = jnp.where(c2, zero, w)
        c1 = (s1 <= 0.0) & (s2 <= 0.0)
        v = jnp.where(c1, zero, v); w = jnp.where(c1, zero, w)
        dvx = apx - v * abx - w * acx
        dvy = apy - v * aby - w * acy
        dvz = apz - v * abz - w * acz
        dd = dvx * dvx + dvy * dvy + dvz * dvz
        fvec = lane + t * 128
        dd = jnp.where(fvec < F, dd, PADD)
        lt1 = dd < min1
        lt2 = (dd < min2) & (~lt1)
        min2 = jnp.where(lt1, min1, jnp.where(lt2, dd, min2))
        fid2 = jnp.where(lt1, fid1, jnp.where(lt2, fvec, fid2))
        min1 = jnp.where(lt1, dd, min1)
        fid1 = jnp.where(lt1, fvec, fid1)
        return min1, fid1, min2, fid2

    init = (jnp.full((PT, 128), BIG), jnp.zeros((PT, 128), jnp.int32),
            jnp.full((PT, 128), BIG), jnp.zeros((PT, 128), jnp.int32))
    min1, fid1, min2, fid2 = lax.fori_loop(0, NT, body, init)
    m1 = jnp.min(min1, axis=1, keepdims=True)
    sel1 = min1 == m1
    f1 = jnp.min(jnp.where(sel1, fid1, IMAX), axis=1, keepdims=True)
    on1 = fid1 == f1
    cand2 = jnp.where(on1, min2, min1)
    cf2 = jnp.where(on1, fid2, fid1)
    m2 = jnp.min(cand2, axis=1, keepdims=True)
    f2 = jnp.min(jnp.where(cand2 == m2, cf2, IMAX), axis=1, keepdims=True)
    oref[0, 0, :, 0:1] = f1
    oref[0, 0, :, 1:2] = f2


def _top2_faces(coords4, t9):
    return pl.pallas_call(
        _sweep_body,
        grid=(B, S // PT),
        in_specs=[
            pl.BlockSpec((1, 1, PT, 3), lambda b, s: (b, s, 0, 0)),
            pl.BlockSpec((1, NT, 9, 128), lambda b, s: (b, 0, 0, 0)),
        ],
        out_specs=pl.BlockSpec((1, 1, PT, 2), lambda b, s: (b, s, 0, 0)),
        out_shape=jax.ShapeDtypeStruct((B, S // PT, PT, 2), jnp.int32),
    )(coords4, t9)


def _cand_gather_body(f1_h, f2_h, c0_h, c1_h, c2_h, svx_h, svy_h, svz_h,
                      cvx_h, cvy_h, cvz_h, *refs):
    outs = refs[:36]
    (f1v, f2v, c0v, c1v, c2v, svx, svy, svz, cvx, cvy, cvz) = refs[36:47]
    fb = refs[47:]
    w = _wid()
    base = w * PCH
    boff = (w // 16) * V
    pltpu.sync_copy(f1_h.at[pl.ds(base, PCH)], f1v)
    pltpu.sync_copy(f2_h.at[pl.ds(base, PCH)], f2v)
    pltpu.sync_copy(c0_h, c0v)
    pltpu.sync_copy(c1_h, c1v)
    pltpu.sync_copy(c2_h, c2v)
    pltpu.sync_copy(svx_h, svx)
    pltpu.sync_copy(svy_h, svy)
    pltpu.sync_copy(svz_h, svz)
    pltpu.sync_copy(cvx_h, cvx)
    pltpu.sync_copy(cvy_h, cvy)
    pltpu.sync_copy(cvz_h, cvz)
    for g in range(PCH // 16):
        sl = pl.ds(g * 16, 16)
        for k, fv in ((0, f1v), (1, f2v)):
            fk = fv[sl]
            i0 = plsc.load_gather(c0v, [fk])
            i1 = plsc.load_gather(c1v, [fk])
            i2 = plsc.load_gather(c2v, [fk])
            a0 = i0 + boff
            a1 = i1 + boff
            a2 = i2 + boff
            r = k * 18
            fb[r + 0][sl] = plsc.load_gather(svx, [a0])
            fb[r + 1][sl] = plsc.load_gather(svy, [a0])
            fb[r + 2][sl] = plsc.load_gather(svz, [a0])
            fb[r + 3][sl] = plsc.load_gather(svx, [a1])
            fb[r + 4][sl] = plsc.load_gather(svy, [a1])
            fb[r + 5][sl] = plsc.load_gather(svz, [a1])
            fb[r + 6][sl] = plsc.load_gather(svx, [a2])
            fb[r + 7][sl] = plsc.load_gather(svy, [a2])
            fb[r + 8][sl] = plsc.load_gather(svz, [a2])
            fb[r + 9][sl] = plsc.load_gather(cvx, [i0])
            fb[r + 10][sl] = plsc.load_gather(cvy, [i0])
            fb[r + 11][sl] = plsc.load_gather(cvz, [i0])
            fb[r + 12][sl] = plsc.load_gather(cvx, [i1])
            fb[r + 13][sl] = plsc.load_gather(cvy, [i1])
            fb[r + 14][sl] = plsc.load_gather(cvz, [i1])
            fb[r + 15][sl] = plsc.load_gather(cvx, [i2])
            fb[r + 16][sl] = plsc.load_gather(cvy, [i2])
            fb[r + 17][sl] = plsc.load_gather(cvz, [i2])
    for r in range(36):
        pltpu.sync_copy(fb[r], outs[r].at[pl.ds(base, PCH)])


@functools.cache
def _cand_gather_kernel():
    return pl.kernel(
        _cand_gather_body,
        mesh=_sc_mesh(),
        compiler_params=pltpu.CompilerParams(needs_layout_passes=False),
        out_type=[jax.ShapeDtypeStruct((BS,), jnp.float32)] * 36,
        scratch_types=[
            pltpu.VMEM((PCH,), jnp.int32),
            pltpu.VMEM((PCH,), jnp.int32),
            pltpu.VMEM((FP,), jnp.int32),
            pltpu.VMEM((FP,), jnp.int32),
            pltpu.VMEM((FP,), jnp.int32),
            pltpu.VMEM((B * V,), jnp.float32),
            pltpu.VMEM((B * V,), jnp.float32),
            pltpu.VMEM((B * V,), jnp.float32),
            pltpu.VMEM((V,), jnp.float32),
            pltpu.VMEM((V,), jnp.float32),
            pltpu.VMEM((V,), jnp.float32),
        ] + [pltpu.VMEM((PCH,), jnp.float32)] * 36,
    )


def _cand_gather(*args):
    return _cand_gather_kernel()(*args)


def _sdiv(x, y):
    return x / jnp.where(jnp.abs(y) > 1e-12, y, 1.0)


def _eric_verbatim(px, py, pz, tb, r0):
    ax = tb[r0 + 0:r0 + 1]; ay = tb[r0 + 1:r0 + 2]; az = tb[r0 + 2:r0 + 3]
    bx = tb[r0 + 3:r0 + 4]; by = tb[r0 + 4:r0 + 5]; bz = tb[r0 + 5:r0 + 6]
    cx = tb[r0 + 6:r0 + 7]; cy = tb[r0 + 7:r0 + 8]; cz = tb[r0 + 8:r0 + 9]
    abx = bx - ax; aby = by - ay; abz = bz - az
    acx = cx - ax; acy = cy - ay; acz = cz - az
    apx = px - ax; apy = py - ay; apz = pz - az
    d1 = abx * apx + aby * apy + abz * apz
    d2 = acx * apx + acy * apy + acz * apz
    bpx = px - bx; bpy = py - by; bpz = pz - bz
    d3 = abx * bpx + aby * bpy + abz * bpz
    d4 = acx * bpx + acy * bpy + acz * bpz
    cpx = px - cx; cpy = py - cy; cpz = pz - cz
    d5 = abx * cpx + aby * cpy + abz * cpz
    d6 = acx * cpx + acy * cpy + acz * cpz
    vc = d1 * d4 - d3 * d2
    vb = d5 * d2 - d1 * d6
    va = d3 * d6 - d5 * d4
    denom = va + vb + vc
    v_i = _sdiv(vb, denom)
    w_i = _sdiv(vc, denom)
    u = 1.0 - v_i - w_i
    v = v_i
    w = w_i
    t6 = _sdiv(d4 - d3, (d4 - d3) + (d5 - d6))
    c6 = (va <= 0.0) & ((d4 - d3) >= 0.0) & ((d5 - d6) >= 0.0)
    u = jnp.where(c6, 0.0, u); v = jnp.where(c6, 1.0 - t6, v)
    w = jnp.where(c6, t6, w)
    t5 = _sdiv(d2, d2 - d6)
    c5 = (vb <= 0.0) & (d2 >= 0.0) & (d6 <= 0.0)
    u = jnp.where(c5, 1.0 - t5, u); v = jnp.where(c5, 0.0, v)
    w = jnp.where(c5, t5, w)
    t4 = _sdiv(d1, d1 - d3)
    c4 = (vc <= 0.0) & (d1 >= 0.0) & (d3 <= 0.0)
    u = jnp.where(c4, 1.0 - t4, u); v = jnp.where(c4, t4, v)
    w = jnp.where(c4, 0.0, w)
    c3 = (d6 >= 0.0) & (d5 <= d6)
    u = jnp.where(c3, 0.0, u); v = jnp.where(c3, 0.0, v)
    w = jnp.where(c3, 1.0, w)
    c2 = (d3 >= 0.0) & (d4 <= d3)
    u = jnp.where(c2, 0.0, u); v = jnp.where(c2, 1.0, v)
    w = jnp.where(c2, 0.0, w)
    c1 = (d1 <= 0.0) & (d2 <= 0.0)
    u = jnp.where(c1, 1.0, u); v = jnp.where(c1, 0.0, v)
    w = jnp.where(c1, 0.0, w)
    cptx = ax * u + bx * v + cx * w
    cpty = ay * u + by * v + cy * w
    cptz = az * u + bz * v + cz * w
    dx = px - cptx; dy = py - cpty; dz = pz - cptz
    dd = dx * dx + dy * dy + dz * dz
    return u, v, w, dd, cptx, cpty, cptz


def _final_body(gref, fref, cref, oref):
    tb = gref[...]
    px = cref[0:1]; py = cref[1:2]; pz = cref[2:3]
    f1 = fref[0:1]; f2 = fref[1:2]
    u1, v1, w1, dd1, x1, y1, z1 = _eric_verbatim(px, py, pz, tb, 0)
    u2, v2, w2, dd2, x2, y2, z2 = _eric_verbatim(px, py, pz, tb, 18)
    dax = px - x1; day = py - y1; daz = pz - z1
    dbx = px - x2; dby = py - y2; dbz = pz - z2
    delta = ((dax - dbx) * (dax + dbx) + (day - dby) * (day + dby)
             + (daz - dbz) * (daz + dbz))
    awin = delta <= 0.0
    u = jnp.where(awin, u1, u2); v = jnp.where(awin, v1, v2)
    w = jnp.where(awin, w1, w2)
    dd = jnp.where(awin, dd1, dd2)
    hx = jnp.where(awin, x1, x2); hy = jnp.where(awin, y1, y2)
    hz = jnp.where(awin, z1, z2)
    cax = jnp.where(awin, tb[9:10], tb[27:28])
    cay = jnp.where(awin, tb[10:11], tb[28:29])
    caz = jnp.where(awin, tb[11:12], tb[29:30])
    cbx = jnp.where(awin, tb[12:13], tb[30:31])
    cby = jnp.where(awin, tb[13:14], tb[31:32])
    cbz = jnp.where(awin, tb[14:15], tb[32:33])
    ccx = jnp.where(awin, tb[15:16], tb[33:34])
    ccy = jnp.where(awin, tb[16:17], tb[34:35])
    ccz = jnp.where(awin, tb[17:18], tb[35:36])
    ox = cax * u + cbx * v + ccx * w
    oy = cay * u + cby * v + ccy * w
    oz = caz * u + cbz * v + ccz * w
    sdf = jnp.sqrt(jnp.maximum(dd, 1e-12))
    dx = hx - px; dy = hy - py; dz = hz - pz
    nrm = jnp.sqrt(dx * dx + dy * dy + dz * dz)
    nd = jnp.maximum(nrm, 1e-6)
    oref[0:1] = ox
    oref[1:2] = oy
    oref[2:3] = oz
    oref[3:4] = sdf
    oref[4:5] = dx / nd
    oref[5:6] = dy / nd
    oref[6:7] = dz / nd
    oref[7:8] = nrm


def _finalize(g40, fids, ct):
    tl = 1024
    return pl.pallas_call(
        _final_body,
        grid=(BS // tl,),
        in_specs=[
            pl.BlockSpec((40, tl), lambda i: (0, i)),
            pl.BlockSpec((8, tl), lambda i: (0, i)),
            pl.BlockSpec((8, tl), lambda i: (0, i)),
        ],
        out_specs=pl.BlockSpec((8, tl), lambda i: (0, i)),
        out_shape=jax.ShapeDtypeStruct((8, BS), jnp.float32),
    )(g40, fids, ct)


def kernel(coords, smpl_V, smpl_F, can_V):
    coords = coords.astype(jnp.float32)
    smpl_V = smpl_V.astype(jnp.float32)
    can_V = can_V.astype(jnp.float32)
    sfp = jnp.pad(smpl_F, ((0, FP - F), (0, 0)))
    c0 = sfp[:, 0].astype(jnp.int32)
    c1 = sfp[:, 1].astype(jnp.int32)
    c2 = sfp[:, 2].astype(jnp.int32)
    svx = smpl_V[:, :, 0].reshape(B * V)
    svy = smpl_V[:, :, 1].reshape(B * V)
    svz = smpl_V[:, :, 2].reshape(B * V)
    cvx = can_V[:, 0]
    cvy = can_V[:, 1]
    cvz = can_V[:, 2]

    tri_rows = _tri_gather(c0, c1, c2, svx, svy, svz)
    tri9 = jnp.stack(tri_rows).reshape(B, 9, FP)
    t9 = tri9.reshape(B, 9, NT, 128).transpose(0, 2, 1, 3)
    coords4 = coords.reshape(B, S // PT, PT, 3)
    fid2 = _top2_faces(coords4, t9)
    fflat = fid2.reshape(BS, 2)
    f1 = fflat[:, 0]
    f2 = fflat[:, 1]
    g_rows = _cand_gather(f1, f2, c0, c1, c2, svx, svy, svz, cvx, cvy, cvz)
    g40 = jnp.concatenate(
        [jnp.stack(g_rows), jnp.zeros((4, BS), jnp.float32)], axis=0)
    fids = jnp.zeros((8, BS), jnp.int32).at[0].set(f1).at[1].set(f2)
    ct = jnp.zeros((8, BS), jnp.float32)
    ct = ct.at[0].set(coords[..., 0].reshape(BS))
    ct = ct.at[1].set(coords[..., 1].reshape(BS))
    ct = ct.at[2].set(coords[..., 2].reshape(BS))
    out8 = _finalize(g40, fids, ct)
    out_coord = out8[0:3].T.reshape(B, S, 3)
    sdf = out8[3].reshape(B, S)
    normal = out8[4:7].T.reshape(B, S, 3)
    z = coords[..., 2:3]
    return (out_coord, sdf, normal, z)
